# baseline (speedup 1.0000x reference)
import jax
import jax.numpy as jnp
from jax.experimental import pallas as pl
from jax.experimental.pallas import tpu as pltpu

N_ROWS = 4096
N_COLS = 100000
ROWS_B = 256
SCOLS = 2048
NBUF = 8
RSUB = 64
N_RB = N_ROWS // ROWS_B
ALIGN_COLS = (N_COLS // 128) * 128
TAIL = N_COLS - ALIGN_COLS
N_CBLK = (ALIGN_COLS + SCOLS - 1) // SCOLS
NEG_INF = float("-inf")
SCALING = 0.5


def _blk_w(c):
    return SCOLS if c < N_CBLK - 1 else ALIGN_COLS - (N_CBLK - 1) * SCOLS


def _copy(x_hbm, row0, c, buf_ref, sem):
    w = _blk_w(c)
    dst = buf_ref if w == SCOLS else buf_ref.at[:, pl.ds(0, w)]
    return pltpu.make_async_copy(
        x_hbm.at[pl.ds(row0, ROWS_B), pl.ds(c * SCOLS, w)],
        dst,
        sem,
    )


def _merge(p1, p2, xk):
    return jnp.maximum(p1, xk), jnp.maximum(p2, jnp.minimum(p1, xk))


def _sweep(buf_ref, p1_ref, p2_ref, c):
    w = _blk_w(c)
    for r in range(0, ROWS_B, RSUB):
        rows = pl.ds(r, RSUB)
        p1 = p1_ref[rows, :]
        p2 = p2_ref[rows, :]
        for k in range(w // 128):
            xk = buf_ref[rows, pl.ds(k * 128, 128)]
            p1, p2 = _merge(p1, p2, xk)
        p1_ref[rows, :] = p1
        p2_ref[rows, :] = p2


def _body(lip_ref, x_hbm, tail_ref, o_ref, *refs):
    bufs = refs[:NBUF]
    sems = refs[NBUF]
    p1_ref, p2_ref = refs[NBUF + 1:]
    i = pl.program_id(0)
    row0 = i * ROWS_B

    p1_ref[...] = jnp.full((ROWS_B, 128), NEG_INF, jnp.float32)
    p2_ref[...] = jnp.full((ROWS_B, 128), NEG_INF, jnp.float32)

    for c in range(min(NBUF, N_CBLK)):
        _copy(x_hbm, row0, c, bufs[c % NBUF], sems.at[c % NBUF]).start()
    for c in range(N_CBLK):
        b = c % NBUF
        _copy(x_hbm, row0, c, bufs[b], sems.at[b]).wait()
        _sweep(bufs[b], p1_ref, p2_ref, c)
        nxt = c + NBUF
        if nxt < N_CBLK:
            _copy(x_hbm, row0, nxt, bufs[b], sems.at[b]).start()

    lane = jax.lax.broadcasted_iota(jnp.int32, (1, 128), 1)
    for r in range(0, ROWS_B, RSUB):
        rows = pl.ds(r, RSUB)
        xt = jnp.where(lane < TAIL, tail_ref[rows, :], NEG_INF)
        p1, p2 = _merge(p1_ref[rows, :], p2_ref[rows, :], xt)
        p1_ref[rows, :] = p1
        p2_ref[rows, :] = p2

    pp1 = p1_ref[...]
    pp2 = p2_ref[...]
    m1 = jnp.max(pp1, axis=1, keepdims=True)
    eq = pp1 == m1
    cnt = jnp.sum(eq.astype(jnp.int32), axis=1, keepdims=True)
    runner = jnp.max(jnp.where(eq, NEG_INF, pp1), axis=1, keepdims=True)
    second_p1 = jnp.where(cnt > 1, m1, runner)
    m2 = jnp.maximum(second_p1, jnp.max(pp2, axis=1, keepdims=True))
    o_ref[...] = jnp.sum(m1 - m2).reshape(1, 1, 1)


def kernel(lipschitz, prediction, target):
    del target
    lip = lipschitz.reshape(1, 1)
    sums = pl.pallas_call(
        _body,
        grid=(N_RB,),
        in_specs=[
            pl.BlockSpec(memory_space=pltpu.SMEM),
            pl.BlockSpec(memory_space=pl.ANY),
            pl.BlockSpec((ROWS_B, 128), lambda i: (i, ALIGN_COLS // 128)),
        ],
        out_specs=pl.BlockSpec((1, 1, 1), lambda i: (i, 0, 0)),
        out_shape=jax.ShapeDtypeStruct((N_RB, 1, 1), jnp.float32),
        scratch_shapes=[pltpu.VMEM((ROWS_B, SCOLS), jnp.float32)] * NBUF
        + [
            pltpu.SemaphoreType.DMA((NBUF,)),
            pltpu.VMEM((ROWS_B, 128), jnp.float32),
            pltpu.VMEM((ROWS_B, 128), jnp.float32),
        ],
        compiler_params=pltpu.CompilerParams(
            dimension_semantics=("parallel",),
        ),
    )(lip, prediction, prediction)
    return (jnp.sum(sums) / N_ROWS) * SCALING / lipschitz
